# speedup vs baseline: 2.9308x; 2.9308x over previous
import functools

import jax
import jax.numpy as jnp
from jax import lax
from jax.experimental import pallas as pl
from jax.experimental.pallas import tpu as pltpu
from jax.experimental.pallas import tpu_sc as plsc

_EPS = 1e-5
_NC = 2
_NS = 16
_NW = _NC * _NS
_CHUNK = 128


def _ln_table(table_ref, gamma_ref, beta_ref, out_ref):
    t = table_ref[...]
    mean = jnp.mean(t, axis=1, keepdims=True)
    cen = t - mean
    var = jnp.mean(cen * cen, axis=1, keepdims=True)
    out_ref[...] = cen * lax.rsqrt(var + _EPS) * gamma_ref[...] + beta_ref[...]


def kernel(task_id, batch_size, table, gamma, beta):
    V, D = table.shape
    B = task_id.shape[0]

    normed = pl.pallas_call(
        _ln_table,
        out_shape=jax.ShapeDtypeStruct((V, D), jnp.float32),
    )(table, gamma.reshape(1, D), beta.reshape(1, D))

    rows_per_w = B // _NW
    n_chunks = rows_per_w // _CHUNK
    idx2d = task_id.astype(jnp.int32).reshape(B // _CHUNK, _CHUNK)

    mesh = plsc.VectorSubcoreMesh(core_axis_name="c", subcore_axis_name="s")

    @functools.partial(
        pl.kernel,
        mesh=mesh,
        out_type=jax.ShapeDtypeStruct((B, D), jnp.float32),
        scratch_types=[
            pltpu.VMEM((n_chunks, _CHUNK), jnp.int32),
            pltpu.VMEM((rows_per_w, D), jnp.float32),
            pltpu.SemaphoreType.DMA,
        ],
    )
    def _gather(idx_hbm, tab_hbm, out_hbm, idx_v, rows_v, sem):
        wid = lax.axis_index("s") * _NC + lax.axis_index("c")
        pltpu.sync_copy(idx_hbm.at[pl.ds(wid * n_chunks, n_chunks)], idx_v)
        copies = [
            pltpu.async_copy(
                tab_hbm.at[idx_v.at[c]],
                rows_v.at[pl.ds(c * _CHUNK, _CHUNK)],
                sem,
            )
            for c in range(n_chunks)
        ]
        for cp in copies:
            cp.wait()
        pltpu.sync_copy(rows_v, out_hbm.at[pl.ds(wid * rows_per_w, rows_per_w)])

    return _gather(idx2d, normed)
